# baseline (speedup 1.0000x reference)
import jax
import jax.numpy as jnp
from jax import lax
from jax.experimental import pallas as pl
from jax.experimental.pallas import tpu as pltpu
from jax.experimental.pallas import tpu_sc as plsc

VOCAB = 100000
HID = 768
B = 4
S = 4096
N = B * S
EPS = 1e-12

NC = 2
NS = 16
NW = NC * NS
TPW = N // NW
C = 32
NCH = TPW // C
HC = HID // 16
G4 = C // 4

_GDN = lax.GatherDimensionNumbers(
    offset_dims=(), collapsed_slice_dims=(0,), start_index_map=(0,))


def _lane_perm(x, perm):
    return lax.gather(x, perm[:, None], _GDN, (1,),
                      mode=lax.GatherScatterMode.PROMISE_IN_BOUNDS)


def _butterfly_sum(x):
    lanes = jnp.arange(16, dtype=jnp.int32)
    for k in (8, 4, 2, 1):
        x = x + _lane_perm(x, lanes ^ k)
    return x


def _rsqrt_nr(v):
    vi = lax.bitcast_convert_type(v, jnp.int32)
    y = lax.bitcast_convert_type(jnp.int32(0x5F3759DF) - (vi >> 1),
                                 jnp.float32)
    y = y * (1.5 - 0.5 * v * y * y)
    y = y * (1.5 - 0.5 * v * y * y)
    y = y * (1.5 - 0.5 * v * y * y)
    return y


def _body(ids, tts, pids, wtab, ttab, ptab, gam, bet, out,
          idx_v, tt_v, pidx_v, ttf_v,
          xw0, xp0, xw1, xp1, obuf,
          t0v, dv, gv, bv, sbuf,
          semg0, semg1, semo):
    wid = lax.axis_index("s") * NC + lax.axis_index("c")
    base = pl.multiple_of(wid * TPW, TPW)
    pltpu.sync_copy(ids.at[pl.ds(base, TPW)], idx_v)
    pltpu.sync_copy(tts.at[pl.ds(base, TPW)], tt_v)
    pltpu.sync_copy(pids.at[pl.ds(base, TPW)], pidx_v)
    pltpu.sync_copy(gam, gv)
    pltpu.sync_copy(bet, bv)
    pltpu.sync_copy(ttab, obuf.at[pl.ds(0, 2)])
    for j in range(HC):
        sl = pl.ds(j * 16, 16)
        r0 = obuf[0, sl]
        t0v[sl] = r0
        dv[sl] = obuf[1, sl] - r0

    def conv_body(k, carry):
        o = pl.multiple_of(k * 16, 16)
        ttf_v[pl.ds(o, 16)] = tt_v[pl.ds(o, 16)].astype(jnp.float32)
        return carry

    lax.fori_loop(0, TPW // 16, conv_body, 0)

    bufs = ((xw0, xp0, semg0), (xw1, xp1, semg1))

    def issue_gathers(g, xw, xp, semg):
        off = pl.multiple_of(g * C, C)
        pltpu.async_copy(wtab.at[idx_v.at[pl.ds(off, C)]], xw, semg)
        pltpu.async_copy(ptab.at[pidx_v.at[pl.ds(off, C)]], xp, semg)

    issue_gathers(0, xw0, xp0, semg0)

    def pair_body(i, carry):
        for par in (0, 1):
            g = i * 2 + par
            xw_c, xp_c, semg_c = bufs[par]
            xw_n, xp_n, semg_n = bufs[1 - par]

            @pl.when(g + 1 < NCH)
            def _():
                issue_gathers(g + 1, xw_n, xp_n, semg_n)

            pltpu.make_async_copy(out.at[pl.ds(0, C)], xw_c, semg_c).wait()
            pltpu.make_async_copy(out.at[pl.ds(0, C)], xp_c, semg_c).wait()

            off = pl.multiple_of(g * C, C)

            def pass1_body(gi, carry2):
                t0 = gi * 4
                ab = pl.multiple_of(off + ((t0 // 16) * 16), 16)
                lane0 = t0 % 16
                w16 = ttf_v[pl.ds(ab, 16)]
                for k in range(4):
                    t = t0 + k
                    mult = _lane_perm(
                        w16, jnp.full((16,), lane0 + k, jnp.int32))
                    acc = jnp.zeros((16,), jnp.float32)
                    acc2 = jnp.zeros((16,), jnp.float32)
                    for j in range(HC):
                        sl = pl.ds(j * 16, 16)
                        x = (xw_c[t, sl] + xp_c[t, sl]
                             + (t0v[sl] + mult * dv[sl]))
                        xw_c[t, sl] = x
                        acc = acc + x
                        acc2 = acc2 + x * x
                    mv = _butterfly_sum(acc) * (1.0 / HID)
                    v = _butterfly_sum(acc2) * (1.0 / HID) - mv * mv + EPS
                    sbuf[pl.ds(t * 16, 16)] = mv
                    sbuf[pl.ds(C * 16 + t * 16, 16)] = _rsqrt_nr(v)
                return carry2

            lax.fori_loop(0, G4, pass1_body, 0)

            @pl.when(g >= 1)
            def _():
                pltpu.make_async_copy(obuf, out.at[pl.ds(0, C)],
                                      semo).wait()

            def pass2_body(gi, carry2):
                t0 = gi * 4
                stats = [(sbuf[pl.ds((t0 + k) * 16, 16)],
                          sbuf[pl.ds(C * 16 + (t0 + k) * 16, 16)])
                         for k in range(4)]
                for j in range(HC):
                    sl = pl.ds(j * 16, 16)
                    gj = gv[sl]
                    bj = bv[sl]
                    for k in range(4):
                        t = t0 + k
                        mv, y = stats[k]
                        obuf[t, sl] = (xw_c[t, sl] - mv) * y * gj + bj
                return carry2

            lax.fori_loop(0, G4, pass2_body, 0)
            pltpu.async_copy(obuf, out.at[pl.ds(base + off, C)], semo)
        return carry

    lax.fori_loop(0, NCH // 2, pair_body, 0)
    pltpu.make_async_copy(obuf, out.at[pl.ds(0, C)], semo).wait()


def kernel(input_ids, token_type_ids, turn_type_ids, word_table, type_table,
           pos_table, ln_gamma, ln_beta):
    ids = input_ids.reshape(-1)
    tts = token_type_ids.reshape(-1)
    pids = turn_type_ids.reshape(-1)
    mesh = plsc.VectorSubcoreMesh(core_axis_name="c", subcore_axis_name="s")
    f = pl.kernel(
        _body,
        out_type=jax.ShapeDtypeStruct((N, HID), jnp.float32),
        mesh=mesh,
        scratch_types=[
            pltpu.VMEM((TPW,), jnp.int32),
            pltpu.VMEM((TPW,), jnp.int32),
            pltpu.VMEM((TPW,), jnp.int32),
            pltpu.VMEM((TPW,), jnp.float32),
            pltpu.VMEM((C, HID), jnp.float32),
            pltpu.VMEM((C, HID), jnp.float32),
            pltpu.VMEM((C, HID), jnp.float32),
            pltpu.VMEM((C, HID), jnp.float32),
            pltpu.VMEM((C, HID), jnp.float32),
            pltpu.VMEM((HID,), jnp.float32),
            pltpu.VMEM((HID,), jnp.float32),
            pltpu.VMEM((HID,), jnp.float32),
            pltpu.VMEM((HID,), jnp.float32),
            pltpu.VMEM((2 * C * 16,), jnp.float32),
            pltpu.SemaphoreType.DMA,
            pltpu.SemaphoreType.DMA,
            pltpu.SemaphoreType.DMA,
        ],
    )
    out = f(ids, tts, pids, word_table, type_table, pos_table, ln_gamma, ln_beta)
    return out.reshape(B, S, HID)


# speedup vs baseline: 1.4999x; 1.4999x over previous
import jax
import jax.numpy as jnp
from jax import lax
from jax.experimental import pallas as pl
from jax.experimental.pallas import tpu as pltpu
from jax.experimental.pallas import tpu_sc as plsc

VOCAB = 100000
HID = 768
B = 4
S = 4096
N = B * S
EPS = 1e-12

NC = 2
NS = 16
NW = NC * NS
TPW = N // NW
C = 32
NCH = TPW // C
HC = HID // 16
G4 = C // 4

_GDN = lax.GatherDimensionNumbers(
    offset_dims=(), collapsed_slice_dims=(0,), start_index_map=(0,))


def _lane_perm(x, perm):
    return lax.gather(x, perm[:, None], _GDN, (1,),
                      mode=lax.GatherScatterMode.PROMISE_IN_BOUNDS)


def _butterfly_sum(x):
    lanes = jnp.arange(16, dtype=jnp.int32)
    for k in (8, 4, 2, 1):
        x = x + _lane_perm(x, lanes ^ k)
    return x


def _rsqrt_nr(v):
    vi = lax.bitcast_convert_type(v, jnp.int32)
    y = lax.bitcast_convert_type(jnp.int32(0x5F3759DF) - (vi >> 1),
                                 jnp.float32)
    y = y * (1.5 - 0.5 * v * y * y)
    y = y * (1.5 - 0.5 * v * y * y)
    y = y * (1.5 - 0.5 * v * y * y)
    return y


def _body(ids, tts, pids, wtab, ttab, ptab, gam, bet, out,
          idx_v, tt_v, pidx_v,
          xw0, xp0, xw1, xp1, obuf,
          t0v, dv, gv, bv, sbuf,
          semg0, semg1, semo):
    wid = lax.axis_index("s") * NC + lax.axis_index("c")
    base = pl.multiple_of(wid * TPW, TPW)
    pltpu.sync_copy(ids.at[pl.ds(base, TPW)], idx_v)
    pltpu.sync_copy(tts.at[pl.ds(base, TPW)], tt_v)
    pltpu.sync_copy(pids.at[pl.ds(base, TPW)], pidx_v)
    pltpu.sync_copy(gam, gv)
    pltpu.sync_copy(bet, bv)
    pltpu.sync_copy(ttab, obuf.at[pl.ds(0, 2)])
    for j in range(HC):
        sl = pl.ds(j * 16, 16)
        r0 = obuf[0, sl]
        t0v[sl] = r0
        dv[sl] = obuf[1, sl] - r0

    bufs = ((xw0, xp0, semg0), (xw1, xp1, semg1))

    def issue_gathers(g, xw, xp, semg):
        off = pl.multiple_of(g * C, C)
        pltpu.async_copy(wtab.at[idx_v.at[pl.ds(off, C)]], xw, semg)
        pltpu.async_copy(ptab.at[pidx_v.at[pl.ds(off, C)]], xp, semg)

    issue_gathers(0, xw0, xp0, semg0)

    def pair_body(i, carry):
        for par in (0, 1):
            g = i * 2 + par
            xw_c, xp_c, semg_c = bufs[par]
            xw_n, xp_n, semg_n = bufs[1 - par]

            @pl.when(g + 1 < NCH)
            def _():
                issue_gathers(g + 1, xw_n, xp_n, semg_n)

            pltpu.make_async_copy(out.at[pl.ds(0, C)], xw_c, semg_c).wait()
            pltpu.make_async_copy(out.at[pl.ds(0, C)], xp_c, semg_c).wait()

            off = pl.multiple_of(g * C, C)

            def pass1_body(gi, carry2):
                t0 = gi * 2
                ab = pl.multiple_of(off + ((t0 // 16) * 16), 16)
                lane0 = t0 % 16
                w16 = tt_v[pl.ds(ab, 16)]
                mults = [_lane_perm(w16, jnp.full((16,), lane0 + k,
                                                  jnp.int32))
                         .astype(jnp.float32)
                         for k in range(2)]
                accs = [jnp.zeros((16,), jnp.float32) for _ in range(2)]
                acc2s = [jnp.zeros((16,), jnp.float32) for _ in range(2)]
                for j in range(HC):
                    sl = pl.ds(j * 16, 16)
                    tv = t0v[sl]
                    dj = dv[sl]
                    for k in range(2):
                        t = t0 + k
                        x = (xw_c[t, sl] + xp_c[t, sl]
                             + (tv + mults[k] * dj))
                        xw_c[t, sl] = x
                        accs[k] = accs[k] + x
                        acc2s[k] = acc2s[k] + x * x
                for k in range(2):
                    t = t0 + k
                    mv = _butterfly_sum(accs[k]) * (1.0 / HID)
                    v = (_butterfly_sum(acc2s[k]) * (1.0 / HID)
                         - mv * mv + EPS)
                    sbuf[pl.ds(t * 16, 16)] = mv
                    sbuf[pl.ds(C * 16 + t * 16, 16)] = _rsqrt_nr(v)
                return carry2

            lax.fori_loop(0, C // 2, pass1_body, 0)

            @pl.when(g >= 1)
            def _():
                pltpu.make_async_copy(obuf, out.at[pl.ds(0, C)],
                                      semo).wait()

            def pass2_body(gi, carry2):
                t0 = gi * 4
                stats = [(sbuf[pl.ds((t0 + k) * 16, 16)],
                          sbuf[pl.ds(C * 16 + (t0 + k) * 16, 16)])
                         for k in range(4)]
                for j in range(HC):
                    sl = pl.ds(j * 16, 16)
                    gj = gv[sl]
                    bj = bv[sl]
                    for k in range(4):
                        t = t0 + k
                        mv, y = stats[k]
                        obuf[t, sl] = (xw_c[t, sl] - mv) * y * gj + bj
                return carry2

            lax.fori_loop(0, G4, pass2_body, 0)
            pltpu.async_copy(obuf, out.at[pl.ds(base + off, C)], semo)
        return carry

    lax.fori_loop(0, NCH // 2, pair_body, 0)
    pltpu.make_async_copy(obuf, out.at[pl.ds(0, C)], semo).wait()


def kernel(input_ids, token_type_ids, turn_type_ids, word_table, type_table,
           pos_table, ln_gamma, ln_beta):
    ids = input_ids.reshape(-1)
    tts = token_type_ids.reshape(-1)
    pids = turn_type_ids.reshape(-1)
    mesh = plsc.VectorSubcoreMesh(core_axis_name="c", subcore_axis_name="s")
    f = pl.kernel(
        _body,
        out_type=jax.ShapeDtypeStruct((N, HID), jnp.float32),
        mesh=mesh,
        scratch_types=[
            pltpu.VMEM((TPW,), jnp.int32),
            pltpu.VMEM((TPW,), jnp.int32),
            pltpu.VMEM((TPW,), jnp.int32),
            pltpu.VMEM((C, HID), jnp.float32),
            pltpu.VMEM((C, HID), jnp.float32),
            pltpu.VMEM((C, HID), jnp.float32),
            pltpu.VMEM((C, HID), jnp.float32),
            pltpu.VMEM((C, HID), jnp.float32),
            pltpu.VMEM((HID,), jnp.float32),
            pltpu.VMEM((HID,), jnp.float32),
            pltpu.VMEM((HID,), jnp.float32),
            pltpu.VMEM((HID,), jnp.float32),
            pltpu.VMEM((2 * C * 16,), jnp.float32),
            pltpu.SemaphoreType.DMA,
            pltpu.SemaphoreType.DMA,
            pltpu.SemaphoreType.DMA,
        ],
    )
    out = f(ids, tts, pids, word_table, type_table, pos_table, ln_gamma, ln_beta)
    return out.reshape(B, S, HID)


# speedup vs baseline: 5.0785x; 3.3858x over previous
import jax
import jax.numpy as jnp
from jax import lax
from jax.experimental import pallas as pl
from jax.experimental.pallas import tpu as pltpu
from jax.experimental.pallas import tpu_sc as plsc

VOCAB = 100000
HID = 768
B = 4
S = 4096
N = B * S
EPS = 1e-12

NC = 2
NS = 16
NW = NC * NS
TPW = N // NW
C = 32
NCH = TPW // C
HC = HID // 16

TB = 512


def _sc_body(ids, pids, wtab, ptab, out,
             idx_v, pidx_v, xw0, xp0, xw1, xp1, obuf,
             semg0, semg1, semo):
    wid = lax.axis_index("s") * NC + lax.axis_index("c")
    base = pl.multiple_of(wid * TPW, TPW)
    pltpu.sync_copy(ids.at[pl.ds(base, TPW)], idx_v)
    pltpu.sync_copy(pids.at[pl.ds(base, TPW)], pidx_v)

    bufs = ((xw0, xp0, semg0), (xw1, xp1, semg1))

    def issue_gathers(g, xw, xp, semg):
        off = pl.multiple_of(g * C, C)
        pltpu.async_copy(wtab.at[idx_v.at[pl.ds(off, C)]], xw, semg)
        pltpu.async_copy(ptab.at[pidx_v.at[pl.ds(off, C)]], xp, semg)

    issue_gathers(0, xw0, xp0, semg0)

    def pair_body(i, carry):
        for par in (0, 1):
            g = i * 2 + par
            xw_c, xp_c, semg_c = bufs[par]
            xw_n, xp_n, semg_n = bufs[1 - par]

            @pl.when(g + 1 < NCH)
            def _():
                issue_gathers(g + 1, xw_n, xp_n, semg_n)

            pltpu.make_async_copy(out.at[pl.ds(0, C)], xw_c, semg_c).wait()
            pltpu.make_async_copy(out.at[pl.ds(0, C)], xp_c, semg_c).wait()

            @pl.when(g >= 1)
            def _():
                pltpu.make_async_copy(obuf, out.at[pl.ds(0, C)],
                                      semo).wait()

            off = pl.multiple_of(g * C, C)

            def sum_body(ti, carry2):
                for k in range(2):
                    t = ti * 2 + k
                    for j in range(HC):
                        sl = pl.ds(j * 16, 16)
                        obuf[t, sl] = xw_c[t, sl] + xp_c[t, sl]
                return carry2

            lax.fori_loop(0, C // 2, sum_body, 0)
            pltpu.async_copy(obuf, out.at[pl.ds(base + off, C)], semo)
        return carry

    lax.fori_loop(0, NCH // 2, pair_body, 0)
    pltpu.make_async_copy(obuf, out.at[pl.ds(0, C)], semo).wait()


def _tc_ln_body(x_ref, tt_ref, ttab_ref, g_ref, b_ref, o_ref):
    x = x_ref[...]
    mf = tt_ref[0, 0, :].astype(jnp.float32)
    t0 = ttab_ref[0, :]
    d = ttab_ref[1, :] - t0
    x = x + t0[None, :] + mf[:, None] * d[None, :]
    mean = jnp.mean(x, axis=-1, keepdims=True)
    xc = x - mean
    var = jnp.mean(xc * xc, axis=-1, keepdims=True)
    y = xc * lax.rsqrt(var + EPS)
    o_ref[...] = y * g_ref[0, :][None, :] + b_ref[0, :][None, :]


def kernel(input_ids, token_type_ids, turn_type_ids, word_table, type_table,
           pos_table, ln_gamma, ln_beta):
    ids = input_ids.reshape(-1)
    tts = token_type_ids.reshape(-1)
    pids = turn_type_ids.reshape(-1)

    mesh = plsc.VectorSubcoreMesh(core_axis_name="c", subcore_axis_name="s")
    sc = pl.kernel(
        _sc_body,
        out_type=jax.ShapeDtypeStruct((N, HID), jnp.float32),
        mesh=mesh,
        scratch_types=[
            pltpu.VMEM((TPW,), jnp.int32),
            pltpu.VMEM((TPW,), jnp.int32),
            pltpu.VMEM((C, HID), jnp.float32),
            pltpu.VMEM((C, HID), jnp.float32),
            pltpu.VMEM((C, HID), jnp.float32),
            pltpu.VMEM((C, HID), jnp.float32),
            pltpu.VMEM((C, HID), jnp.float32),
            pltpu.SemaphoreType.DMA,
            pltpu.SemaphoreType.DMA,
            pltpu.SemaphoreType.DMA,
        ],
    )
    x = sc(ids, pids, word_table, pos_table)

    nb = N // TB
    out = pl.pallas_call(
        _tc_ln_body,
        out_shape=jax.ShapeDtypeStruct((N, HID), jnp.float32),
        grid=(nb,),
        in_specs=[
            pl.BlockSpec((TB, HID), lambda i: (i, 0)),
            pl.BlockSpec((1, 1, TB), lambda i: (i, 0, 0)),
            pl.BlockSpec((2, HID), lambda i: (0, 0)),
            pl.BlockSpec((1, HID), lambda i: (0, 0)),
            pl.BlockSpec((1, HID), lambda i: (0, 0)),
        ],
        out_specs=pl.BlockSpec((TB, HID), lambda i: (i, 0)),
    )(x, tts.reshape(nb, 1, TB), type_table, ln_gamma.reshape(1, HID),
      ln_beta.reshape(1, HID))
    return out.reshape(B, S, HID)
